# Initial kernel scaffold; baseline (speedup 1.0000x reference)
import jax
import jax.numpy as jnp
from jax.experimental import pallas as pl


def kernel(bitstrings, kernel, indices, mask):
    raise NotImplementedError("write your pallas kernel here")


# speedup vs baseline: 12.3475x; 12.3475x over previous
import jax
import jax.numpy as jnp
from jax import lax
from jax.experimental import pallas as pl

NUM_BITS = 64


def _tc_body(bits_ref, kv_ref, idx_ref, msk_ref, out_ref):
    idx = idx_ref[...]
    msk = msk_ref[...]
    kv = kv_ref[...]
    T = idx.shape[0]

    iota = lax.broadcasted_iota(jnp.int32, (T, NUM_BITS), 1)
    e0 = (idx[:, 0:1] == iota).astype(jnp.float32)
    e1 = (idx[:, 1:2] == iota).astype(jnp.float32)
    a0 = (msk[:, 0:1] > 0).astype(jnp.float32)
    a1 = (msk[:, 1:2] > 0).astype(jnp.float32)

    w_pair = kv * a0 * a1
    w_lin0 = kv * a0 * (1.0 - a1)
    w_lin1 = kv * (1.0 - a0) * a1
    w_const = kv * (1.0 - a0) * (1.0 - a1)

    dn = (((0,), (0,)), ((), ()))
    W = lax.dot_general(e0 * w_pair, e1, dn,
                        precision=lax.Precision.HIGHEST,
                        preferred_element_type=jnp.float32)
    h0 = lax.dot_general(e0, w_lin0, dn,
                         precision=lax.Precision.HIGHEST,
                         preferred_element_type=jnp.float32)
    h1 = lax.dot_general(e1, w_lin1, dn,
                         precision=lax.Precision.HIGHEST,
                         preferred_element_type=jnp.float32)
    h = (h0 + h1).reshape(1, NUM_BITS)
    c = jnp.sum(w_const)

    spins = (1 - 2 * bits_ref[...]).astype(jnp.float32)
    sw = jnp.dot(spins, W, precision=lax.Precision.HIGHEST,
                 preferred_element_type=jnp.float32)
    out_ref[...] = jnp.sum((sw + h) * spins, axis=1, keepdims=True) + c


def kernel(bitstrings, kernel, indices, mask):
    B = bitstrings.shape[0]
    kv = kernel.reshape(-1, 1)
    out = pl.pallas_call(
        _tc_body,
        out_shape=jax.ShapeDtypeStruct((B, 1), jnp.float32),
    )(bitstrings, kv, indices, mask)
    return out.reshape(B)
